# baseline (speedup 1.0000x reference)
import functools

import jax
import jax.numpy as jnp
from jax import lax
from jax.experimental import pallas as pl
from jax.experimental.pallas import tpu as pltpu
from jax.experimental.pallas import tpu_sc as plsc

_W = 640
_L = 16
_CH = 128
_CROWS = 40


def kernel(indices, table):
    (B,) = indices.shape
    V, D = table.shape

    info = plsc.get_sparse_core_info()
    NC, NS = info.num_cores, info.num_subcores
    NW = NC * NS

    n_tcols = (V + 127) // 128
    per_w = (n_tcols + NW - 1) // NW
    slab = per_w * 128
    n_blocks = slab // _W
    v_main = (V // 128) * 128
    tail_w = V - v_main
    last_start = v_main - _W
    cap = B + _L
    n_chunks = (B + _CH - 1) // _CH + 1

    idx = indices.astype(jnp.int32)
    tableT = table.T

    mesh = plsc.VectorSubcoreMesh(core_axis_name="c", subcore_axis_name="s")

    @functools.partial(
        pl.kernel,
        mesh=mesh,
        out_type=(
            jax.ShapeDtypeStruct((NW, n_chunks * _CROWS, _CH), jnp.float32),
            jax.ShapeDtypeStruct((NW, 8, _CH), jnp.int32),
        ),
        scratch_types=[
            pltpu.VMEM((cap,), jnp.int32),
            pltpu.VMEM((cap,), jnp.int32),
            pltpu.VMEM((cap,), jnp.int32),
            pltpu.VMEM((cap,), jnp.int32),
            pltpu.VMEM((2, D, _W), jnp.float32),
            pltpu.VMEM((D, tail_w), jnp.float32),
            pltpu.VMEM((_CROWS, _CH), jnp.float32),
            pltpu.VMEM((1, _CH), jnp.int32),
            pltpu.SemaphoreType.DMA,
            pltpu.SemaphoreType.DMA,
        ],
        compiler_params=pltpu.CompilerParams(needs_layout_passes=False),
    )
    def sweep(idx_hbm, table_hbm, pout_hbm, cnt_hbm, idxbuf, hitidx, hitb,
              sortedb, blkbuf, tailbuf, acc, cntbuf, sem, semt):
        c = lax.axis_index("c")
        s = lax.axis_index("s")
        wid = s * NC + c
        mylo = wid * slab
        myhi = jnp.minimum(mylo + slab, V)

        lane = lax.iota(jnp.int32, _L)
        lane_hi = lane + _L

        def blk_start(j):
            return jnp.minimum(mylo + j * _W, last_start)

        descs = {}
        for j in range(2):
            descs[j] = pltpu.async_copy(
                table_hbm.at[:, pl.ds(pl.multiple_of(blk_start(j), 128), _W)],
                blkbuf.at[j % 2],
                sem,
            )
        tail_desc = pltpu.async_copy(
            table_hbm.at[:, pl.ds(v_main, tail_w)], tailbuf, semt
        )

        pltpu.sync_copy(idx_hbm.at[pl.ds(0, B)], idxbuf.at[pl.ds(0, B)])

        def scan(v, off):
            vec = idxbuf[pl.ds(v * _L, _L)]
            mask = (vec >= mylo) & (vec < myhi)
            plsc.store_compressed(hitidx.at[pl.ds(off, _L)], vec, mask=mask)
            bvec = lane + v * _L
            plsc.store_compressed(hitb.at[pl.ds(off, _L)], bvec, mask=mask)
            cnt = jnp.max(plsc.all_reduce_population_count(mask))
            return off + cnt

        nhits = lax.fori_loop(0, B // _L, scan, jnp.int32(0))
        nvec = (nhits + _L - 1) // _L

        seg = [jnp.int32(0)]
        for j in range(n_blocks):
            sel_lo = mylo + j * _W
            sel_hi = blk_start(j) + _W

            def binb(t, off, sel_lo=sel_lo, sel_hi=sel_hi):
                hv = hitidx[pl.ds(t * _L, _L)]
                bv = hitb[pl.ds(t * _L, _L)]
                valid = (t * _L + lane) < nhits
                mask = valid & (hv >= sel_lo) & (hv < sel_hi)
                plsc.store_compressed(idxbuf.at[pl.ds(off, _L)], hv, mask=mask)
                plsc.store_compressed(sortedb.at[pl.ds(off, _L)], bv,
                                      mask=mask)
                cnt = jnp.max(plsc.all_reduce_population_count(mask))
                return off + cnt

            seg.append(lax.fori_loop(0, nvec, binb, seg[-1]))

        def bint(t, off):
            hv = hitidx[pl.ds(t * _L, _L)]
            bv = hitb[pl.ds(t * _L, _L)]
            valid = (t * _L + lane) < nhits
            mask = valid & (hv >= v_main)
            plsc.store_compressed(idxbuf.at[pl.ds(off, _L)], hv, mask=mask)
            plsc.store_compressed(sortedb.at[pl.ds(off, _L)], bv, mask=mask)
            cnt = jnp.max(plsc.all_reduce_population_count(mask))
            return off + cnt

        seg.append(lax.fori_loop(0, nvec, bint, seg[-1]))

        def flush(chunk):
            for g in range(_CH // _L):
                tags = sortedb[pl.ds(chunk * _CH + g * _L, _L)]
                pos = chunk * _CH + g * _L + lane
                tags = jnp.where(pos < nhits, tags, B + g * _L + lane)
                acc[D, pl.ds(g * _L, _L)] = plsc.bitcast(tags, jnp.float32)
            pl.delay(200)
            pltpu.sync_copy(
                acc,
                pout_hbm.at[wid, pl.ds(
                    pl.multiple_of(chunk * _CROWS, 8), _CROWS), :],
            )

        def extract_from(buf_ref, base):
            def ext(t, carry):
                iv = idxbuf[pl.ds(t, _L)][0]
                slot = jnp.full((_L,), t % _CH, jnp.int32)
                col = jnp.full((_L,), iv - base, jnp.int32)
                v0 = plsc.load_gather(buf_ref, [lane, col])
                v1 = plsc.load_gather(buf_ref, [lane_hi, col])
                plsc.store_scatter(acc, [lane, slot], v0)
                plsc.store_scatter(acc, [lane_hi, slot], v1)

                @pl.when(t % _CH == _CH - 1)
                def _():
                    flush(t // _CH)

                return carry

            return ext

        for j in range(n_blocks):
            descs[j].wait()
            lax.fori_loop(
                seg[j], seg[j + 1],
                extract_from(blkbuf.at[j % 2], blk_start(j)), 0)
            nxt = j + 2
            if nxt < n_blocks:
                descs[nxt] = pltpu.async_copy(
                    table_hbm.at[
                        :, pl.ds(pl.multiple_of(blk_start(nxt), 128), _W)],
                    blkbuf.at[nxt % 2],
                    sem,
                )
        tail_desc.wait()
        lax.fori_loop(
            seg[n_blocks], seg[n_blocks + 1],
            extract_from(tailbuf, jnp.int32(v_main)), 0)

        @pl.when(nhits % _CH != 0)
        def _():
            flush(nhits // _CH)

        def wcnt(g, carry):
            cntbuf[0, pl.ds(g * _L, _L)] = jnp.full((_L,), nhits, jnp.int32)
            return carry

        lax.fori_loop(0, _CH // _L, wcnt, 0)
        pltpu.sync_copy(cntbuf, cnt_hbm.at[wid, pl.ds(0, 1), :])

    @functools.partial(
        pl.kernel,
        mesh=mesh,
        out_type=jax.ShapeDtypeStruct((B + _CH, D), jnp.float32),
        scratch_types=[
            pltpu.VMEM((_CROWS, _CH), jnp.float32),
            pltpu.VMEM((_CH, D), jnp.float32),
            pltpu.VMEM((_CH,), jnp.int32),
            pltpu.VMEM((1, _CH), jnp.int32),
            pltpu.SemaphoreType.DMA,
        ],
        compiler_params=pltpu.CompilerParams(
            use_tc_tiling_on_sc=False, needs_layout_passes=False),
    )
    def scatter(pout_hbm, cnt_hbm, out_hbm, chunk, rows, tags, cntv, sem):
        c = lax.axis_index("c")
        s = lax.axis_index("s")
        wid = s * NC + c
        lane = lax.iota(jnp.int32, _L)
        pltpu.sync_copy(cnt_hbm.at[wid, pl.ds(0, 1), :], cntv)
        nhits = cntv[0, pl.ds(0, _L)][0]
        nck = (nhits + _CH - 1) // _CH

        def body(ck, carry):
            pltpu.sync_copy(
                pout_hbm.at[wid, pl.ds(ck * _CROWS, _CROWS), :], chunk)
            for g in range(_CH // _L):
                tags[pl.ds(g * _L, _L)] = plsc.bitcast(
                    chunk[D, pl.ds(g * _L, _L)], jnp.int32)
            for k in range(D):
                for g in range(_CH // _L):
                    slot = g * _L + lane
                    vals = plsc.load_gather(
                        chunk, [jnp.full((_L,), k, jnp.int32), slot])
                    plsc.store_scatter(
                        rows, [slot, jnp.full((_L,), k, jnp.int32)], vals)
            pltpu.async_copy(rows, out_hbm.at[tags], sem).wait()
            return carry

        lax.fori_loop(0, nck, body, 0)

    pout, cnts = sweep(idx, tableT)
    out = scatter(pout, cnts)
    return out[:B]
